# Initial kernel scaffold; baseline (speedup 1.0000x reference)
import jax
import jax.numpy as jnp
from jax.experimental import pallas as pl


def kernel(x, edge_index, W1, b1, W2, b2):
    raise NotImplementedError("write your pallas kernel here")


# speedup vs baseline: 11.7604x; 11.7604x over previous
import functools

import jax
import jax.numpy as jnp
from jax import lax
from jax.experimental import pallas as pl
from jax.experimental.pallas import tpu as pltpu
from jax.experimental.pallas import tpu_sc as plsc

N = 10000
D = 128
E = 320000

NC = 2
NS = 16
CHUNK = 128
CPT = 80
E_TILE = CPT * CHUNK
E_PAD = NC * NS * E_TILE
N_PAD = 10240
STRIPE = N_PAD // NS

_mesh = plsc.VectorSubcoreMesh(core_axis_name="c", subcore_axis_name="s")


def _scatter_body(p_hbm, src_hbm, dst_hbm, zeros_hbm, out_hbm,
                  sidx_v, didx_v, rows_v, sem, acc_sh):
    cid = lax.axis_index("c")
    sid = lax.axis_index("s")
    base = (cid * NS + sid) * E_TILE

    pltpu.sync_copy(zeros_hbm, rows_v)
    for q in range(STRIPE // CHUNK):
        pltpu.sync_copy(
            rows_v, acc_sh.at[pl.ds(STRIPE * sid + CHUNK * q, CHUNK)])
    plsc.subcore_barrier()

    def _chunk(j, _):
        e0 = base + j * CHUNK
        pltpu.sync_copy(src_hbm.at[pl.ds(e0, CHUNK)], sidx_v)
        pltpu.sync_copy(dst_hbm.at[pl.ds(e0, CHUNK)], didx_v)
        pltpu.async_copy(p_hbm.at[sidx_v], rows_v, sem).wait()
        pltpu.sync_copy(rows_v, acc_sh.at[didx_v], add=True)
        return 0

    lax.fori_loop(0, CPT, _chunk, 0)
    plsc.subcore_barrier()

    for q in range(STRIPE // CHUNK):
        pltpu.sync_copy(
            acc_sh.at[pl.ds(STRIPE * sid + CHUNK * q, CHUNK)], rows_v)
        pltpu.sync_copy(
            rows_v,
            out_hbm.at[pl.ds(cid * N_PAD + STRIPE * sid + CHUNK * q, CHUNK)])


_scatter_kernel = functools.partial(
    pl.kernel,
    out_type=jax.ShapeDtypeStruct((NC * N_PAD, D), jnp.float32),
    mesh=_mesh,
    scratch_types=[
        pltpu.VMEM((CHUNK,), jnp.int32),
        pltpu.VMEM((CHUNK,), jnp.int32),
        pltpu.VMEM((CHUNK, D), jnp.float32),
        pltpu.SemaphoreType.DMA,
        pltpu.VMEM_SHARED((N_PAD, D), jnp.float32),
    ],
)(_scatter_body)


_RB = 512
_HIGH = lax.Precision.HIGHEST


def _scale_mm_body(x_ref, deg_ref, w_ref, p_ref):
    d = deg_ref[...]
    g = lax.rsqrt(d[0, :, 0:1] + d[1, :, 0:1] + 1.0)
    p_ref[...] = jnp.dot(
        x_ref[...], w_ref[...],
        precision=_HIGH, preferred_element_type=jnp.float32) * g


def _mid_body(acc_ref, p_ref, deg_ref, b1_ref, w_ref, p2_ref):
    d = deg_ref[...]
    g = lax.rsqrt(d[0, :, 0:1] + d[1, :, 0:1] + 1.0)
    a = acc_ref[...]
    h = g * (a[0] + a[1] + p_ref[...]) + b1_ref[...]
    h = jnp.maximum(h, 0.0)
    p2_ref[...] = jnp.dot(
        h, w_ref[...],
        precision=_HIGH, preferred_element_type=jnp.float32) * g


def _final_body(acc_ref, p_ref, deg_ref, b2_ref, o_ref):
    d = deg_ref[...]
    g = lax.rsqrt(d[0, :, 0:1] + d[1, :, 0:1] + 1.0)
    a = acc_ref[...]
    o = g * (a[0] + a[1] + p_ref[...]) + b2_ref[...]
    m = jnp.max(o, axis=1, keepdims=True)
    e = jnp.exp(o - m)
    o_ref[...] = (o - m) - jnp.log(jnp.sum(e, axis=1, keepdims=True))


_row_spec = pl.BlockSpec((_RB, D), lambda i: (i, 0))
_acc_spec = pl.BlockSpec((NC, _RB, D), lambda i: (0, i, 0))
_w_spec = pl.BlockSpec((D, D), lambda i: (0, 0))
_b_spec = pl.BlockSpec((1, D), lambda i: (0, 0))
_grid = (N_PAD // _RB,)
_row_out = jax.ShapeDtypeStruct((N_PAD, D), jnp.float32)

_scale_mm = pl.pallas_call(
    _scale_mm_body, grid=_grid,
    in_specs=[_row_spec, _acc_spec, _w_spec],
    out_specs=_row_spec, out_shape=_row_out)

_mid = pl.pallas_call(
    _mid_body, grid=_grid,
    in_specs=[_acc_spec, _row_spec, _acc_spec, _b_spec, _w_spec],
    out_specs=_row_spec, out_shape=_row_out)

_final = pl.pallas_call(
    _final_body, grid=_grid,
    in_specs=[_acc_spec, _row_spec, _acc_spec, _b_spec],
    out_specs=_row_spec, out_shape=_row_out)


def kernel(x, edge_index, W1, b1, W2, b2):
    src = edge_index[0].astype(jnp.int32)
    dst = edge_index[1].astype(jnp.int32)
    pad = N + (jnp.arange(E_PAD - E, dtype=jnp.int32) % (N_PAD - N))
    src_p = jnp.concatenate([src, pad])
    dst_p = jnp.concatenate([dst, pad])
    x_p = jnp.pad(x, ((0, N_PAD - N), (0, 0)))
    b1r = b1.reshape(1, D)
    b2r = b2.reshape(1, D)
    zeros_c = jnp.zeros((CHUNK, D), jnp.float32)
    ones_t = jnp.ones((N_PAD, D), jnp.float32)

    deg = _scatter_kernel(ones_t, dst_p, dst_p, zeros_c).reshape(NC, N_PAD, D)
    p1 = _scale_mm(x_p, deg, W1)
    acc1 = _scatter_kernel(p1, src_p, dst_p, zeros_c).reshape(NC, N_PAD, D)
    p2 = _mid(acc1, p1, deg, b1r, W2)
    acc2 = _scatter_kernel(p2, src_p, dst_p, zeros_c).reshape(NC, N_PAD, D)
    out = _final(acc2, p2, deg, b2r)
    return out[:N]


# speedup vs baseline: 25.0856x; 2.1331x over previous
import functools

import jax
import jax.numpy as jnp
from jax import lax
from jax.experimental import pallas as pl
from jax.experimental.pallas import tpu as pltpu
from jax.experimental.pallas import tpu_sc as plsc

N = 10000
D = 128
E = 320000

NC = 2
NS = 16
CHUNK = 128
G = 40
NG = 2
CPT = G * NG
E_TILE = CPT * CHUNK
E_PAD = NC * NS * E_TILE
N_PAD = 10240
STRIPE = N_PAD // NS

_mesh = plsc.VectorSubcoreMesh(core_axis_name="c", subcore_axis_name="s")


def _scatter_body(p_hbm, srcg_hbm, dstg_hbm, zeros_hbm, out_hbm,
                  src_v, dst_v, rows_v, sem0, sem1, acc_sh):
    cid = lax.axis_index("c")
    sid = lax.axis_index("s")
    b = cid * NS + sid

    pltpu.sync_copy(zeros_hbm, rows_v.at[0])
    for q in range(STRIPE // CHUNK):
        pltpu.sync_copy(
            rows_v.at[0], acc_sh.at[pl.ds(STRIPE * sid + CHUNK * q, CHUNK)])
    plsc.subcore_barrier()

    sems = (sem0, sem1)

    def _group(gg, _):
        pltpu.sync_copy(srcg_hbm.at[b * NG + gg], src_v)
        pltpu.sync_copy(dstg_hbm.at[b * NG + gg], dst_v)
        descs = [
            pltpu.async_copy(p_hbm.at[src_v.at[0]], rows_v.at[0], sem0),
            pltpu.async_copy(p_hbm.at[src_v.at[1]], rows_v.at[1], sem1),
        ]
        for j in range(G):
            h = j % 2
            descs[h].wait()
            pltpu.sync_copy(rows_v.at[h], acc_sh.at[dst_v.at[j]], add=True)
            if j + 2 < G:
                descs[h] = pltpu.async_copy(
                    p_hbm.at[src_v.at[j + 2]], rows_v.at[h], sems[h])
        return 0

    lax.fori_loop(0, NG, _group, 0)
    plsc.subcore_barrier()

    for q in range(STRIPE // CHUNK):
        pltpu.sync_copy(
            acc_sh.at[pl.ds(STRIPE * sid + CHUNK * q, CHUNK)], rows_v.at[0])
        pltpu.sync_copy(
            rows_v.at[0],
            out_hbm.at[pl.ds(cid * N_PAD + STRIPE * sid + CHUNK * q, CHUNK)])


_scatter_kernel = functools.partial(
    pl.kernel,
    out_type=jax.ShapeDtypeStruct((NC * N_PAD, D), jnp.float32),
    mesh=_mesh,
    scratch_types=[
        pltpu.VMEM((G, CHUNK), jnp.int32),
        pltpu.VMEM((G, CHUNK), jnp.int32),
        pltpu.VMEM((2, CHUNK, D), jnp.float32),
        pltpu.SemaphoreType.DMA,
        pltpu.SemaphoreType.DMA,
        pltpu.VMEM_SHARED((N_PAD, D), jnp.float32),
    ],
)(_scatter_body)


def _deg_body(ones_hbm, dstg_hbm, zeros_hbm, out_hbm, dst_v, rows_v, acc_sh):
    cid = lax.axis_index("c")
    sid = lax.axis_index("s")
    b = cid * NS + sid

    pltpu.sync_copy(zeros_hbm, rows_v)
    for q in range(STRIPE // CHUNK):
        pltpu.sync_copy(
            rows_v, acc_sh.at[pl.ds(STRIPE * sid + CHUNK * q, CHUNK)])
    plsc.subcore_barrier()

    pltpu.sync_copy(ones_hbm, rows_v)

    def _group(gg, _):
        pltpu.sync_copy(dstg_hbm.at[b * NG + gg], dst_v)
        for j in range(G):
            pltpu.sync_copy(rows_v, acc_sh.at[dst_v.at[j]], add=True)
        return 0

    lax.fori_loop(0, NG, _group, 0)
    plsc.subcore_barrier()

    for q in range(STRIPE // CHUNK):
        pltpu.sync_copy(
            acc_sh.at[pl.ds(STRIPE * sid + CHUNK * q, CHUNK)], rows_v)
        pltpu.sync_copy(
            rows_v,
            out_hbm.at[pl.ds(cid * N_PAD + STRIPE * sid + CHUNK * q, CHUNK)])


_deg_kernel = functools.partial(
    pl.kernel,
    out_type=jax.ShapeDtypeStruct((NC * N_PAD, D), jnp.float32),
    mesh=_mesh,
    scratch_types=[
        pltpu.VMEM((G, CHUNK), jnp.int32),
        pltpu.VMEM((CHUNK, D), jnp.float32),
        pltpu.VMEM_SHARED((N_PAD, D), jnp.float32),
    ],
)(_deg_body)


_RB = 512
_HIGH = lax.Precision.HIGHEST


def _scale_mm_body(x_ref, deg_ref, w_ref, p_ref):
    d = deg_ref[...]
    g = lax.rsqrt(d[0, :, 0:1] + d[1, :, 0:1] + 1.0)
    p_ref[...] = jnp.dot(
        x_ref[...], w_ref[...],
        precision=_HIGH, preferred_element_type=jnp.float32) * g


def _mid_body(acc_ref, p_ref, deg_ref, b1_ref, w_ref, p2_ref):
    d = deg_ref[...]
    g = lax.rsqrt(d[0, :, 0:1] + d[1, :, 0:1] + 1.0)
    a = acc_ref[...]
    h = g * (a[0] + a[1] + p_ref[...]) + b1_ref[...]
    h = jnp.maximum(h, 0.0)
    p2_ref[...] = jnp.dot(
        h, w_ref[...],
        precision=_HIGH, preferred_element_type=jnp.float32) * g


def _final_body(acc_ref, p_ref, deg_ref, b2_ref, o_ref):
    d = deg_ref[...]
    g = lax.rsqrt(d[0, :, 0:1] + d[1, :, 0:1] + 1.0)
    a = acc_ref[...]
    o = g * (a[0] + a[1] + p_ref[...]) + b2_ref[...]
    m = jnp.max(o, axis=1, keepdims=True)
    e = jnp.exp(o - m)
    o_ref[...] = (o - m) - jnp.log(jnp.sum(e, axis=1, keepdims=True))


_row_spec = pl.BlockSpec((_RB, D), lambda i: (i, 0))
_acc_spec = pl.BlockSpec((NC, _RB, D), lambda i: (0, i, 0))
_w_spec = pl.BlockSpec((D, D), lambda i: (0, 0))
_b_spec = pl.BlockSpec((1, D), lambda i: (0, 0))
_grid = (N_PAD // _RB,)
_row_out = jax.ShapeDtypeStruct((N_PAD, D), jnp.float32)

_scale_mm = pl.pallas_call(
    _scale_mm_body, grid=_grid,
    in_specs=[_row_spec, _acc_spec, _w_spec],
    out_specs=_row_spec, out_shape=_row_out)

_mid = pl.pallas_call(
    _mid_body, grid=_grid,
    in_specs=[_acc_spec, _row_spec, _acc_spec, _b_spec, _w_spec],
    out_specs=_row_spec, out_shape=_row_out)

_final = pl.pallas_call(
    _final_body, grid=_grid,
    in_specs=[_acc_spec, _row_spec, _acc_spec, _b_spec],
    out_specs=_row_spec, out_shape=_row_out)


def kernel(x, edge_index, W1, b1, W2, b2):
    src = edge_index[0].astype(jnp.int32)
    dst = edge_index[1].astype(jnp.int32)
    pad = N + (jnp.arange(E_PAD - E, dtype=jnp.int32) % (N_PAD - N))
    src_g = jnp.concatenate([src, pad]).reshape(NC * NS * NG, G, CHUNK)
    dst_g = jnp.concatenate([dst, pad]).reshape(NC * NS * NG, G, CHUNK)
    x_p = jnp.pad(x, ((0, N_PAD - N), (0, 0)))
    b1r = b1.reshape(1, D)
    b2r = b2.reshape(1, D)
    zeros_c = jnp.zeros((CHUNK, D), jnp.float32)
    ones_c = jnp.ones((CHUNK, D), jnp.float32)

    deg = _deg_kernel(ones_c, dst_g, zeros_c).reshape(NC, N_PAD, D)
    p1 = _scale_mm(x_p, deg, W1)
    acc1 = _scatter_kernel(p1, src_g, dst_g, zeros_c).reshape(NC, N_PAD, D)
    p2 = _mid(acc1, p1, deg, b1r, W2)
    acc2 = _scatter_kernel(p2, src_g, dst_g, zeros_c).reshape(NC, N_PAD, D)
    out = _final(acc2, p2, deg, b2r)
    return out[:N]
